# baseline (device time: 74757 ns/iter reference)
import jax
import jax.numpy as jnp
from jax import lax
from jax.experimental import pallas as pl
from jax.experimental.pallas import tpu as pltpu

N_DEV = 4
T = 512
D = 512
F = 1024
E_LOC = 2


def kernel(x, assign, W1, W2):
    a2d = assign.reshape(T, 1)

    def body(x_ref, a_ref, w1_ref, w2_ref, out_ref,
             x_all, a_all, contrib, comb,
             x_send, x_recv, a_send, a_recv, c_send, c_recv):
        my = lax.axis_index("i")

        barrier = pltpu.get_barrier_semaphore()
        for r in range(1, N_DEV):
            tgt = lax.rem(my + r, N_DEV)
            pl.semaphore_signal(barrier, inc=1, device_id=(tgt,),
                                device_id_type=pl.DeviceIdType.MESH)
        pl.semaphore_wait(barrier, N_DEV - 1)

        def desc(src, dst, row, sems_s, sems_r, r, tgt):
            return pltpu.make_async_remote_copy(
                src_ref=src,
                dst_ref=dst.at[pl.ds(row * T, T), :],
                send_sem=sems_s.at[r - 1],
                recv_sem=sems_r.at[r - 1],
                device_id=(tgt,),
                device_id_type=pl.DeviceIdType.MESH,
            )

        drain = []
        for r in range(1, N_DEV):
            tgt = lax.rem(my + r, N_DEV)
            xs = desc(x_ref, x_all, my, x_send, x_recv, r, tgt)
            sa = desc(a_ref, a_all, my, a_send, a_recv, r, tgt)
            xs.start()
            sa.start()
            drain += [xs, sa]

        x_all[pl.ds(my * T, T), :] = x_ref[...]
        a_all[pl.ds(my * T, T), :] = a_ref[...]

        for r in range(1, N_DEV):
            src_pos = lax.rem(my - r + N_DEV, N_DEV)
            desc(x_ref, x_all, src_pos, x_send, x_recv, r, 0).wait_recv()
            desc(a_ref, a_all, src_pos, a_send, a_recv, r, 0).wait_recv()

        xa = x_all[...]
        aa = a_all[...]
        acc = jnp.zeros((N_DEV * T, D), jnp.float32)
        for k in range(E_LOC):
            eid = my * E_LOC + k
            xm = jnp.where(aa == eid, xa, 0.0)
            h = jnp.maximum(
                jnp.dot(xm, w1_ref[k], preferred_element_type=jnp.float32),
                0.0)
            acc = acc + jnp.dot(h, w2_ref[k],
                                preferred_element_type=jnp.float32)
        contrib[...] = acc

        for r in range(1, N_DEV):
            tgt = lax.rem(my + r, N_DEV)
            cd = pltpu.make_async_remote_copy(
                src_ref=contrib.at[pl.ds(tgt * T, T), :],
                dst_ref=comb.at[pl.ds(my * T, T), :],
                send_sem=c_send.at[r - 1],
                recv_sem=c_recv.at[r - 1],
                device_id=(tgt,),
                device_id_type=pl.DeviceIdType.MESH,
            )
            cd.start()
            drain.append(cd)

        comb[pl.ds(my * T, T), :] = contrib[pl.ds(my * T, T), :]

        for r in range(1, N_DEV):
            src_pos = lax.rem(my - r + N_DEV, N_DEV)
            pltpu.make_async_remote_copy(
                src_ref=contrib.at[pl.ds(0, T), :],
                dst_ref=comb.at[pl.ds(src_pos * T, T), :],
                send_sem=c_send.at[r - 1],
                recv_sem=c_recv.at[r - 1],
                device_id=(0,),
                device_id_type=pl.DeviceIdType.MESH,
            ).wait_recv()

        out_ref[...] = (comb[pl.ds(0 * T, T), :] + comb[pl.ds(1 * T, T), :]
                        + comb[pl.ds(2 * T, T), :] + comb[pl.ds(3 * T, T), :])

        for d in drain:
            d.wait_send()

    return pl.pallas_call(
        body,
        out_shape=jax.ShapeDtypeStruct((T, D), jnp.float32),
        in_specs=[pl.BlockSpec(memory_space=pltpu.VMEM)] * 4,
        out_specs=pl.BlockSpec(memory_space=pltpu.VMEM),
        scratch_shapes=[
            pltpu.VMEM((N_DEV * T, D), jnp.float32),
            pltpu.VMEM((N_DEV * T, 1), jnp.int32),
            pltpu.VMEM((N_DEV * T, D), jnp.float32),
            pltpu.VMEM((N_DEV * T, D), jnp.float32),
            pltpu.SemaphoreType.DMA((N_DEV - 1,)),
            pltpu.SemaphoreType.DMA((N_DEV - 1,)),
            pltpu.SemaphoreType.DMA((N_DEV - 1,)),
            pltpu.SemaphoreType.DMA((N_DEV - 1,)),
            pltpu.SemaphoreType.DMA((N_DEV - 1,)),
            pltpu.SemaphoreType.DMA((N_DEV - 1,)),
        ],
        compiler_params=pltpu.CompilerParams(collective_id=0),
    )(x, a2d, W1, W2)


# device time: 67982 ns/iter; 1.0997x vs baseline; 1.0997x over previous
import jax
import jax.numpy as jnp
from jax import lax
from jax.experimental import pallas as pl
from jax.experimental.pallas import tpu as pltpu

N_DEV = 4
T = 512
D = 512
F = 1024
E_LOC = 2


def kernel(x, assign, W1, W2):
    a2d = assign.reshape(T, 1)

    def body(x_ref, a_ref, w1_ref, w2_ref, out_ref,
             x_all, a_all, contrib, comb,
             x_send, x_recv, a_send, a_recv, c_send, c_recv):
        my = lax.axis_index("i")

        barrier = pltpu.get_barrier_semaphore()
        for r in range(1, N_DEV):
            tgt = lax.rem(my + r, N_DEV)
            pl.semaphore_signal(barrier, inc=1, device_id=(tgt,),
                                device_id_type=pl.DeviceIdType.MESH)
        pl.semaphore_wait(barrier, N_DEV - 1)

        def desc(src, dst, row, sems_s, sems_r, r, tgt):
            return pltpu.make_async_remote_copy(
                src_ref=src,
                dst_ref=dst.at[pl.ds(row * T, T), :],
                send_sem=sems_s.at[r - 1],
                recv_sem=sems_r.at[r - 1],
                device_id=(tgt,),
                device_id_type=pl.DeviceIdType.MESH,
            )

        def compute_chunk(xa, aa):
            acc = jnp.zeros((T, D), jnp.float32)
            for k in range(E_LOC):
                eid = my * E_LOC + k
                xm = jnp.where(aa == eid, xa, 0.0)
                h = jnp.maximum(
                    jnp.dot(xm, w1_ref[k],
                            preferred_element_type=jnp.float32),
                    0.0)
                acc = acc + jnp.dot(h, w2_ref[k],
                                    preferred_element_type=jnp.float32)
            return acc

        drain = []
        for r in range(1, N_DEV):
            tgt = lax.rem(my + r, N_DEV)
            xs = desc(x_ref, x_all, my, x_send, x_recv, r, tgt)
            sa = desc(a_ref, a_all, my, a_send, a_recv, r, tgt)
            xs.start()
            sa.start()
            drain += [xs, sa]

        comb[pl.ds(my * T, T), :] = compute_chunk(x_ref[...], a_ref[...])

        for r in range(1, N_DEV):
            src_pos = lax.rem(my - r + N_DEV, N_DEV)
            desc(x_ref, x_all, src_pos, x_send, x_recv, r, 0).wait_recv()
            desc(a_ref, a_all, src_pos, a_send, a_recv, r, 0).wait_recv()
            contrib[pl.ds(src_pos * T, T), :] = compute_chunk(
                x_all[pl.ds(src_pos * T, T), :],
                a_all[pl.ds(src_pos * T, T), :])
            cd = pltpu.make_async_remote_copy(
                src_ref=contrib.at[pl.ds(src_pos * T, T), :],
                dst_ref=comb.at[pl.ds(my * T, T), :],
                send_sem=c_send.at[r - 1],
                recv_sem=c_recv.at[(N_DEV - r) - 1],
                device_id=(src_pos,),
                device_id_type=pl.DeviceIdType.MESH,
            )
            cd.start()
            drain.append(cd)

        for rc in range(1, N_DEV):
            src_pos = lax.rem(my - rc + N_DEV, N_DEV)
            pltpu.make_async_remote_copy(
                src_ref=contrib.at[pl.ds(0, T), :],
                dst_ref=comb.at[pl.ds(src_pos * T, T), :],
                send_sem=c_send.at[rc - 1],
                recv_sem=c_recv.at[rc - 1],
                device_id=(0,),
                device_id_type=pl.DeviceIdType.MESH,
            ).wait_recv()

        out_ref[...] = (comb[pl.ds(0 * T, T), :] + comb[pl.ds(1 * T, T), :]
                        + comb[pl.ds(2 * T, T), :] + comb[pl.ds(3 * T, T), :])

        for d in drain:
            d.wait_send()

    return pl.pallas_call(
        body,
        out_shape=jax.ShapeDtypeStruct((T, D), jnp.float32),
        in_specs=[pl.BlockSpec(memory_space=pltpu.VMEM)] * 4,
        out_specs=pl.BlockSpec(memory_space=pltpu.VMEM),
        scratch_shapes=[
            pltpu.VMEM((N_DEV * T, D), jnp.float32),
            pltpu.VMEM((N_DEV * T, 1), jnp.int32),
            pltpu.VMEM((N_DEV * T, D), jnp.float32),
            pltpu.VMEM((N_DEV * T, D), jnp.float32),
            pltpu.SemaphoreType.DMA((N_DEV - 1,)),
            pltpu.SemaphoreType.DMA((N_DEV - 1,)),
            pltpu.SemaphoreType.DMA((N_DEV - 1,)),
            pltpu.SemaphoreType.DMA((N_DEV - 1,)),
            pltpu.SemaphoreType.DMA((N_DEV - 1,)),
            pltpu.SemaphoreType.DMA((N_DEV - 1,)),
        ],
        compiler_params=pltpu.CompilerParams(collective_id=0),
    )(x, a2d, W1, W2)


# device time: 44416 ns/iter; 1.6831x vs baseline; 1.5306x over previous
import jax
import jax.numpy as jnp
from jax import lax
from jax.experimental import pallas as pl
from jax.experimental.pallas import tpu as pltpu

N_DEV = 4
T = 512
D = 512
F = 1024
E_LOC = 2


def kernel(x, assign, W1, W2):
    a2d = assign.reshape(T, 1)
    xb = x.astype(jnp.bfloat16)
    W1b = W1.astype(jnp.bfloat16)
    W2b = W2.astype(jnp.bfloat16)

    def body(x_ref, a_ref, w1_ref, w2_ref, out_ref,
             x_all, a_all, contrib, comb,
             x_send, x_recv, a_send, a_recv, c_send, c_recv):
        my = lax.axis_index("i")

        barrier = pltpu.get_barrier_semaphore()
        for r in range(1, N_DEV):
            tgt = lax.rem(my + r, N_DEV)
            pl.semaphore_signal(barrier, inc=1, device_id=(tgt,),
                                device_id_type=pl.DeviceIdType.MESH)
        pl.semaphore_wait(barrier, N_DEV - 1)

        def desc(src, dst, row, sems_s, sems_r, r, tgt):
            return pltpu.make_async_remote_copy(
                src_ref=src,
                dst_ref=dst.at[pl.ds(row * T, T), :],
                send_sem=sems_s.at[r - 1],
                recv_sem=sems_r.at[r - 1],
                device_id=(tgt,),
                device_id_type=pl.DeviceIdType.MESH,
            )

        def compute_chunk(xa, aa):
            acc = jnp.zeros((T, D), jnp.float32)
            for k in range(E_LOC):
                eid = my * E_LOC + k
                xm = jnp.where(aa == eid, xa,
                               jnp.bfloat16(0.0)).astype(jnp.bfloat16)
                h = jnp.maximum(
                    jnp.dot(xm, w1_ref[k],
                            preferred_element_type=jnp.float32),
                    0.0).astype(jnp.bfloat16)
                acc = acc + jnp.dot(h, w2_ref[k],
                                    preferred_element_type=jnp.float32)
            return acc.astype(jnp.bfloat16)

        drain = []
        for r in range(1, N_DEV):
            tgt = lax.rem(my + r, N_DEV)
            xs = desc(x_ref, x_all, my, x_send, x_recv, r, tgt)
            sa = desc(a_ref, a_all, my, a_send, a_recv, r, tgt)
            xs.start()
            sa.start()
            drain += [xs, sa]

        comb[pl.ds(my * T, T), :] = compute_chunk(x_ref[...], a_ref[...])

        for r in range(1, N_DEV):
            src_pos = lax.rem(my - r + N_DEV, N_DEV)
            desc(x_ref, x_all, src_pos, x_send, x_recv, r, 0).wait_recv()
            desc(a_ref, a_all, src_pos, a_send, a_recv, r, 0).wait_recv()
            contrib[pl.ds(src_pos * T, T), :] = compute_chunk(
                x_all[pl.ds(src_pos * T, T), :],
                a_all[pl.ds(src_pos * T, T), :])
            cd = pltpu.make_async_remote_copy(
                src_ref=contrib.at[pl.ds(src_pos * T, T), :],
                dst_ref=comb.at[pl.ds(my * T, T), :],
                send_sem=c_send.at[r - 1],
                recv_sem=c_recv.at[(N_DEV - r) - 1],
                device_id=(src_pos,),
                device_id_type=pl.DeviceIdType.MESH,
            )
            cd.start()
            drain.append(cd)

        for rc in range(1, N_DEV):
            src_pos = lax.rem(my - rc + N_DEV, N_DEV)
            pltpu.make_async_remote_copy(
                src_ref=contrib.at[pl.ds(0, T), :],
                dst_ref=comb.at[pl.ds(src_pos * T, T), :],
                send_sem=c_send.at[rc - 1],
                recv_sem=c_recv.at[rc - 1],
                device_id=(0,),
                device_id_type=pl.DeviceIdType.MESH,
            ).wait_recv()

        out_ref[...] = (
            comb[pl.ds(0 * T, T), :].astype(jnp.float32)
            + comb[pl.ds(1 * T, T), :].astype(jnp.float32)
            + comb[pl.ds(2 * T, T), :].astype(jnp.float32)
            + comb[pl.ds(3 * T, T), :].astype(jnp.float32))

        for d in drain:
            d.wait_send()

    return pl.pallas_call(
        body,
        out_shape=jax.ShapeDtypeStruct((T, D), jnp.float32),
        in_specs=[pl.BlockSpec(memory_space=pltpu.VMEM)] * 4,
        out_specs=pl.BlockSpec(memory_space=pltpu.VMEM),
        scratch_shapes=[
            pltpu.VMEM((N_DEV * T, D), jnp.bfloat16),
            pltpu.VMEM((N_DEV * T, 1), jnp.int32),
            pltpu.VMEM((N_DEV * T, D), jnp.bfloat16),
            pltpu.VMEM((N_DEV * T, D), jnp.bfloat16),
            pltpu.SemaphoreType.DMA((N_DEV - 1,)),
            pltpu.SemaphoreType.DMA((N_DEV - 1,)),
            pltpu.SemaphoreType.DMA((N_DEV - 1,)),
            pltpu.SemaphoreType.DMA((N_DEV - 1,)),
            pltpu.SemaphoreType.DMA((N_DEV - 1,)),
            pltpu.SemaphoreType.DMA((N_DEV - 1,)),
        ],
        compiler_params=pltpu.CompilerParams(collective_id=0),
    )(xb, a2d, W1b, W2b)
